# baseline (speedup 1.0000x reference)
import jax
import jax.numpy as jnp
from jax.experimental import pallas as pl
from jax.experimental.pallas import tpu as pltpu

N = 2048
F = 256
NH = 128
H = 8
P = 2
DH = NH // H
ALPHA = 0.5
RB = 256
RS = 256
AW = 24
LOG2E = 1.4426950408889634


def _proj_kernel(feat_ref, wmat_ref, asrc_ref, adstT_ref, sel_ref, ones_ref,
                 whaug_ref, whlo_ref, esrc_ref, edstT_ref):
    wh = jnp.dot(feat_ref[...], wmat_ref[0],
                 preferred_element_type=jnp.float32)
    whaug = jnp.dot(wh, sel_ref[...],
                    preferred_element_type=jnp.float32) + ones_ref[...]
    hi = whaug.astype(jnp.bfloat16)
    whaug_ref[0] = hi
    whlo_ref[0] = (whaug - hi.astype(jnp.float32)).astype(jnp.bfloat16)
    es = jnp.dot(wh, asrc_ref[0],
                 preferred_element_type=jnp.float32)
    esrc_ref[0] = jnp.concatenate(
        [jnp.exp2(es), jnp.exp2(0.5 * es)], axis=1)
    ed = jax.lax.dot_general(
        adstT_ref[0], wh, (((1,), (1,)), ((), ())),
        preferred_element_type=jnp.float32)
    edstT_ref[0] = jnp.concatenate(
        [jnp.exp2(ed), jnp.exp2(0.5 * ed)], axis=0)


def _att_kernel(adj_ref, whaug_ref, whlo_ref, esrc_ref, edstT_ref, z_ref):
    maskf = jnp.where(adj_ref[0] > 0.0, 1.0, 0.0)
    esrc = esrc_ref[0]
    for h in range(H):
        p = jnp.maximum(
            esrc[:, h:h + 1] * edstT_ref[0, h:h + 1, :],
            esrc[:, H + h:H + h + 1] * edstT_ref[0, H + h:H + h + 1, :])
        p = p * maskf
        pb = p.astype(jnp.bfloat16)
        oext = (jnp.dot(pb, whaug_ref[0, :, h * AW:(h + 1) * AW],
                        preferred_element_type=jnp.float32)
                + jnp.dot(pb, whlo_ref[0, :, h * AW:(h + 1) * AW],
                          preferred_element_type=jnp.float32))
        o = oext[:, :DH] / oext[:, DH:DH + 1]
        z_ref[0, :, h * DH:(h + 1) * DH] = jnp.where(o > 0, o, jnp.exp(o) - 1.0)


def _sem_kernel(z_ref, wsem_ref, bsem_ref, qsem_ref, wsum_ref):
    t = jnp.tanh(jnp.dot(z_ref[0], wsem_ref[...],
                         preferred_element_type=jnp.float32)
                 + bsem_ref[...])
    wsum_ref[0, 0, 0] = jnp.sum(t * qsem_ref[...])


def _score_kernel(beta_ref, rel_ref, z_ref, scores_ref, pen_ref):
    i = pl.program_id(0)
    b0 = beta_ref[0, 0]
    b1 = beta_ref[1, 0]
    rel = rel_ref[...]
    zf_all = b0 * z_ref[0] + b1 * z_ref[1]
    zrow = (b0 * z_ref[0, pl.ds(i * RS, RS), :]
            + b1 * z_ref[1, pl.ds(i * RS, RS), :])
    scores_ref[...] = jax.lax.dot_general(
        zrow * rel, zf_all, (((1,), (1,)), ((), ())),
        preferred_element_type=jnp.float32)
    pen_ref[0, 0] = jnp.sum(rel * rel)


def kernel(features, adjs, W_node, a_src, a_dst, W_sem, b_sem, q_sem, relations):
    f32 = jnp.float32
    wmat = jnp.transpose(W_node, (0, 2, 1, 3)).reshape(P, F, NH)
    eye = jnp.eye(H, dtype=f32)
    A_src = (a_src[:, :, :, None] * eye[:, None, :]).reshape(P, NH, H) * LOG2E
    A_dstT = (eye[:, :, None] * a_dst[:, None, :, :]).reshape(P, H, NH) * LOG2E
    selmat = jnp.zeros((NH, H * AW), f32)
    cols = (jnp.arange(NH) // DH) * AW + (jnp.arange(NH) % DH)
    selmat = selmat.at[jnp.arange(NH), cols].set(1.0)
    onesrow = jnp.zeros((1, H * AW), f32).at[0, jnp.arange(H) * AW + DH].set(1.0)
    bsem2 = b_sem.reshape(1, NH)
    qsem2 = q_sem.reshape(1, NH)
    rel2 = relations.reshape(1, NH)

    whaug_all, whlo_all, esrc_all, edstT_all = pl.pallas_call(
        _proj_kernel,
        grid=(P,),
        in_specs=[
            pl.BlockSpec((N, F), lambda p: (0, 0)),
            pl.BlockSpec((1, F, NH), lambda p: (p, 0, 0)),
            pl.BlockSpec((1, NH, H), lambda p: (p, 0, 0)),
            pl.BlockSpec((1, H, NH), lambda p: (p, 0, 0)),
            pl.BlockSpec((NH, H * AW), lambda p: (0, 0)),
            pl.BlockSpec((1, H * AW), lambda p: (0, 0)),
        ],
        out_specs=[
            pl.BlockSpec((1, N, H * AW), lambda p: (p, 0, 0)),
            pl.BlockSpec((1, N, H * AW), lambda p: (p, 0, 0)),
            pl.BlockSpec((1, N, 2 * H), lambda p: (p, 0, 0)),
            pl.BlockSpec((1, 2 * H, N), lambda p: (p, 0, 0)),
        ],
        out_shape=[
            jax.ShapeDtypeStruct((P, N, H * AW), jnp.bfloat16),
            jax.ShapeDtypeStruct((P, N, H * AW), jnp.bfloat16),
            jax.ShapeDtypeStruct((P, N, 2 * H), f32),
            jax.ShapeDtypeStruct((P, 2 * H, N), f32),
        ],
    )(features, wmat, A_src, A_dstT, selmat, onesrow)

    Z = pl.pallas_call(
        _att_kernel,
        grid=(P, N // RB),
        in_specs=[
            pl.BlockSpec((1, RB, N), lambda p, i: (p, i, 0)),
            pl.BlockSpec((1, N, H * AW), lambda p, i: (p, 0, 0)),
            pl.BlockSpec((1, N, H * AW), lambda p, i: (p, 0, 0)),
            pl.BlockSpec((1, RB, 2 * H), lambda p, i: (p, i, 0)),
            pl.BlockSpec((1, 2 * H, N), lambda p, i: (p, 0, 0)),
        ],
        out_specs=pl.BlockSpec((1, RB, NH), lambda p, i: (p, i, 0)),
        out_shape=jax.ShapeDtypeStruct((P, N, NH), f32),
    )(adjs, whaug_all, whlo_all, esrc_all, edstT_all)

    wsum = pl.pallas_call(
        _sem_kernel,
        grid=(P,),
        in_specs=[
            pl.BlockSpec((1, N, NH), lambda p: (p, 0, 0)),
            pl.BlockSpec((NH, NH), lambda p: (0, 0)),
            pl.BlockSpec((1, NH), lambda p: (0, 0)),
            pl.BlockSpec((1, NH), lambda p: (0, 0)),
        ],
        out_specs=pl.BlockSpec((1, 1, 1), lambda p: (p, 0, 0),
                               memory_space=pltpu.SMEM),
        out_shape=jax.ShapeDtypeStruct((P, 1, 1), f32),
    )(Z, W_sem, bsem2, qsem2)

    beta = jax.nn.softmax(wsum[:, 0, 0] / N).reshape(P, 1)

    scores, pen = pl.pallas_call(
        _score_kernel,
        grid=(N // RS,),
        in_specs=[
            pl.BlockSpec(memory_space=pltpu.SMEM),
            pl.BlockSpec((1, NH), lambda i: (0, 0)),
            pl.BlockSpec((P, N, NH), lambda i: (0, 0, 0)),
        ],
        out_specs=[
            pl.BlockSpec((RS, N), lambda i: (i, 0)),
            pl.BlockSpec((1, 1), lambda i: (0, 0), memory_space=pltpu.SMEM),
        ],
        out_shape=[
            jax.ShapeDtypeStruct((N, N), f32),
            jax.ShapeDtypeStruct((1, 1), f32),
        ],
    )(beta, rel2, Z)

    return scores, pen[0, 0]


# speedup vs baseline: 1.3756x; 1.3756x over previous
import jax
import jax.numpy as jnp
from jax.experimental import pallas as pl
from jax.experimental.pallas import tpu as pltpu

N = 2048
F = 256
NH = 128
H = 8
P = 2
DH = NH // H
ALPHA = 0.5
RB = 256
RS = 256
AW = 24
LOG2E = 1.4426950408889634


def _proj_kernel(feat_ref, wmat_ref, asrc_ref, adstT_ref, sel_ref, ones_ref,
                 whaug_ref, esrc_ref, edstT_ref):
    wh = jnp.dot(feat_ref[...], wmat_ref[0],
                 preferred_element_type=jnp.float32)
    whaug_ref[0] = jnp.dot(wh, sel_ref[...],
                           preferred_element_type=jnp.float32) + ones_ref[...]
    es = jnp.dot(wh, asrc_ref[0],
                 preferred_element_type=jnp.float32)
    esrc_ref[0] = jnp.concatenate(
        [jnp.exp2(es), jnp.exp2(0.5 * es)], axis=1)
    ed = jax.lax.dot_general(
        adstT_ref[0], wh, (((1,), (1,)), ((), ())),
        preferred_element_type=jnp.float32)
    edstT_ref[0] = jnp.concatenate(
        [jnp.exp2(ed), jnp.exp2(0.5 * ed)], axis=0)


def _att_kernel(adj_ref, whaug_ref, esrc_ref, edstT_ref, z_ref):
    maskf = jnp.where(adj_ref[0] > 0.0, 1.0, 0.0)
    esrc = esrc_ref[0]
    for h in range(H):
        p = jnp.maximum(
            esrc[:, h:h + 1] * edstT_ref[0, h:h + 1, :],
            esrc[:, H + h:H + h + 1] * edstT_ref[0, H + h:H + h + 1, :])
        p = p * maskf
        oext = jnp.dot(p, whaug_ref[0, :, h * AW:(h + 1) * AW],
                       preferred_element_type=jnp.float32)
        o = oext[:, :DH] / oext[:, DH:DH + 1]
        z_ref[0, :, h * DH:(h + 1) * DH] = jnp.where(o > 0, o, jnp.exp(o) - 1.0)


def _sem_kernel(z_ref, wsem_ref, bsem_ref, qsem_ref, wsum_ref):
    t = jnp.tanh(jnp.dot(z_ref[0], wsem_ref[...],
                         preferred_element_type=jnp.float32)
                 + bsem_ref[...])
    wsum_ref[0, 0, 0] = jnp.sum(t * qsem_ref[...])


def _score_kernel(beta_ref, rel_ref, z_ref, scores_ref, pen_ref):
    i = pl.program_id(0)
    b0 = beta_ref[0, 0]
    b1 = beta_ref[1, 0]
    rel = rel_ref[...]
    zf_all = b0 * z_ref[0] + b1 * z_ref[1]
    zrow = (b0 * z_ref[0, pl.ds(i * RS, RS), :]
            + b1 * z_ref[1, pl.ds(i * RS, RS), :])
    scores_ref[...] = jax.lax.dot_general(
        zrow * rel, zf_all, (((1,), (1,)), ((), ())),
        preferred_element_type=jnp.float32)
    pen_ref[0, 0] = jnp.sum(rel * rel)


def kernel(features, adjs, W_node, a_src, a_dst, W_sem, b_sem, q_sem, relations):
    f32 = jnp.float32
    wmat = jnp.transpose(W_node, (0, 2, 1, 3)).reshape(P, F, NH)
    eye = jnp.eye(H, dtype=f32)
    A_src = (a_src[:, :, :, None] * eye[:, None, :]).reshape(P, NH, H) * LOG2E
    A_dstT = (eye[:, :, None] * a_dst[:, None, :, :]).reshape(P, H, NH) * LOG2E
    selmat = jnp.zeros((NH, H * AW), f32)
    cols = (jnp.arange(NH) // DH) * AW + (jnp.arange(NH) % DH)
    selmat = selmat.at[jnp.arange(NH), cols].set(1.0)
    onesrow = jnp.zeros((1, H * AW), f32).at[0, jnp.arange(H) * AW + DH].set(1.0)
    bsem2 = b_sem.reshape(1, NH)
    qsem2 = q_sem.reshape(1, NH)
    rel2 = relations.reshape(1, NH)

    whaug_all, esrc_all, edstT_all = pl.pallas_call(
        _proj_kernel,
        grid=(P,),
        in_specs=[
            pl.BlockSpec((N, F), lambda p: (0, 0)),
            pl.BlockSpec((1, F, NH), lambda p: (p, 0, 0)),
            pl.BlockSpec((1, NH, H), lambda p: (p, 0, 0)),
            pl.BlockSpec((1, H, NH), lambda p: (p, 0, 0)),
            pl.BlockSpec((NH, H * AW), lambda p: (0, 0)),
            pl.BlockSpec((1, H * AW), lambda p: (0, 0)),
        ],
        out_specs=[
            pl.BlockSpec((1, N, H * AW), lambda p: (p, 0, 0)),
            pl.BlockSpec((1, N, 2 * H), lambda p: (p, 0, 0)),
            pl.BlockSpec((1, 2 * H, N), lambda p: (p, 0, 0)),
        ],
        out_shape=[
            jax.ShapeDtypeStruct((P, N, H * AW), f32),
            jax.ShapeDtypeStruct((P, N, 2 * H), f32),
            jax.ShapeDtypeStruct((P, 2 * H, N), f32),
        ],
    )(features, wmat, A_src, A_dstT, selmat, onesrow)

    Z = pl.pallas_call(
        _att_kernel,
        grid=(P, N // RB),
        in_specs=[
            pl.BlockSpec((1, RB, N), lambda p, i: (p, i, 0)),
            pl.BlockSpec((1, N, H * AW), lambda p, i: (p, 0, 0)),
            pl.BlockSpec((1, RB, 2 * H), lambda p, i: (p, i, 0)),
            pl.BlockSpec((1, 2 * H, N), lambda p, i: (p, 0, 0)),
        ],
        out_specs=pl.BlockSpec((1, RB, NH), lambda p, i: (p, i, 0)),
        out_shape=jax.ShapeDtypeStruct((P, N, NH), f32),
    )(adjs, whaug_all, esrc_all, edstT_all)

    wsum = pl.pallas_call(
        _sem_kernel,
        grid=(P,),
        in_specs=[
            pl.BlockSpec((1, N, NH), lambda p: (p, 0, 0)),
            pl.BlockSpec((NH, NH), lambda p: (0, 0)),
            pl.BlockSpec((1, NH), lambda p: (0, 0)),
            pl.BlockSpec((1, NH), lambda p: (0, 0)),
        ],
        out_specs=pl.BlockSpec((1, 1, 1), lambda p: (p, 0, 0),
                               memory_space=pltpu.SMEM),
        out_shape=jax.ShapeDtypeStruct((P, 1, 1), f32),
    )(Z, W_sem, bsem2, qsem2)

    beta = jax.nn.softmax(wsum[:, 0, 0] / N).reshape(P, 1)

    scores, pen = pl.pallas_call(
        _score_kernel,
        grid=(N // RS,),
        in_specs=[
            pl.BlockSpec(memory_space=pltpu.SMEM),
            pl.BlockSpec((1, NH), lambda i: (0, 0)),
            pl.BlockSpec((P, N, NH), lambda i: (0, 0, 0)),
        ],
        out_specs=[
            pl.BlockSpec((RS, N), lambda i: (i, 0)),
            pl.BlockSpec((1, 1), lambda i: (0, 0), memory_space=pltpu.SMEM),
        ],
        out_shape=[
            jax.ShapeDtypeStruct((N, N), f32),
            jax.ShapeDtypeStruct((1, 1), f32),
        ],
    )(beta, rel2, Z)

    return scores, pen[0, 0]
